# Initial kernel scaffold; baseline (speedup 1.0000x reference)
import jax
import jax.numpy as jnp
from jax.experimental import pallas as pl


def kernel(ufeat, ifeat, W, fc_w, fc_b, enc_edge_index, enc_etypes, dec_edge_index):
    raise NotImplementedError("write your pallas kernel here")


# speedup vs baseline: 22.4504x; 22.4504x over previous
import jax
import jax.numpy as jnp
from jax import lax
from jax.experimental import pallas as pl
from jax.experimental.pallas import tpu as pltpu
from jax.experimental.pallas import tpu_sc as plsc

NUU = 10000
NII = 10000
EE = 320000
EDD = 100000
DIM = 128
ODIM = 64
RR = 5

NC = 2
NS = 16
NW = NC * NS
CW = 128

TBL = RR * NUU
TBLP = TBL + 48
EP = 327680
ECH = EP // CW
ECH_T = ECH // NS
ECH_W = ECH // NW
EDP = 102400
DCH = EDP // CW
DCH_W = DCH // NW

AGG_P = NUU + 112
AGG_T = AGG_P // NS
K4_BLK = 32

_i32 = jnp.int32
_f32 = jnp.float32


def _mesh():
    return plsc.VectorSubcoreMesh(core_axis_name="c", subcore_axis_name="s")


def _sc_params():
    return pltpu.CompilerParams(needs_layout_passes=False)


def _worker_id():
    return lax.axis_index("s") * NC + lax.axis_index("c")


def _deg_body(src_hbm, dst_hbm, et_hbm, degu_hbm, degi_hbm,
              a_v, e_v, deg_v):
    c = lax.axis_index("c")
    t = lax.axis_index("s")

    def zrow(q, _):
        for g in range(8):
            deg_v[q, pl.ds(g * 16, 16)] = jnp.zeros((16,), _f32)
        return 0
    lax.fori_loop(0, TBLP // 128, zrow, 0)

    base = t * ECH_T

    def run(node_hbm):
        def blk(b, _):
            off = base + b * K4_BLK
            pltpu.sync_copy(node_hbm.at[pl.ds(off, K4_BLK)], a_v)
            pltpu.sync_copy(et_hbm.at[pl.ds(off, K4_BLK)], e_v)
            def row(j, _):
                for g in range(8):
                    sl = pl.ds(g * 16, 16)
                    idx = e_v[j, sl] * NUU + a_v[j, sl]
                    plsc.addupdate_scatter(
                        deg_v, [idx >> 7, idx & 127],
                        jnp.full((16,), 1.0, _f32))
                return 0
            lax.fori_loop(0, K4_BLK, row, 0)
            return 0
        lax.fori_loop(0, ECH_T // K4_BLK, blk, 0)

    @pl.when(c == 0)
    def _():
        run(src_hbm)
        pltpu.sync_copy(deg_v, degu_hbm.at[t])
    @pl.when(c == 1)
    def _():
        run(dst_hbm)
        pltpu.sync_copy(deg_v, degi_hbm.at[t])


def _degrees(src_all, dst_all, et_all):
    return pl.kernel(
        _deg_body,
        out_type=(jax.ShapeDtypeStruct((NS, TBLP // 128, 128), _f32),
                  jax.ShapeDtypeStruct((NS, TBLP // 128, 128), _f32)),
        mesh=_mesh(),
        compiler_params=_sc_params(),
        scratch_types=[
            pltpu.VMEM((K4_BLK, CW), _i32),
            pltpu.VMEM((K4_BLK, CW), _i32),
            pltpu.VMEM((TBLP // 128, 128), _f32),
        ],
    )(src_all, dst_all, et_all)


def _mm_body(u_ref, i_ref, w_ref, mu_ref, mi_ref):
    w = w_ref[0]
    mu_ref[...] = jnp.dot(u_ref[...], w, preferred_element_type=_f32)
    mi_ref[...] = jnp.dot(i_ref[...], w, preferred_element_type=_f32)


def _projections(ufeat, ifeat, W):
    nb = 10
    bs = NUU // nb
    return pl.pallas_call(
        _mm_body,
        grid=(RR, nb),
        in_specs=[
            pl.BlockSpec((bs, DIM), lambda r, i: (i, 0)),
            pl.BlockSpec((bs, DIM), lambda r, i: (i, 0)),
            pl.BlockSpec((1, DIM, DIM), lambda r, i: (r, 0, 0)),
        ],
        out_specs=[
            pl.BlockSpec((bs, DIM), lambda r, i: (r * 10 + i, 0)),
            pl.BlockSpec((bs, DIM), lambda r, i: (r * 10 + i, 0)),
        ],
        out_shape=(jax.ShapeDtypeStruct((TBL, DIM), _f32),
                   jax.ShapeDtypeStruct((TBL, DIM), _f32)),
    )(ufeat, ifeat, W)


def _norm_body(du_ref, di_ref, cu_ref, ci_ref):
    du = jnp.sum(du_ref[...], axis=0)
    di = jnp.sum(di_ref[...], axis=0)
    cu_ref[...] = lax.rsqrt(jnp.maximum(du, 1.0))
    ci_ref[...] = lax.rsqrt(jnp.maximum(di, 1.0))


def _norms(du_parts, di_parts):
    return pl.pallas_call(
        _norm_body,
        out_shape=(jax.ShapeDtypeStruct((TBLP // 128, 128), _f32),
                   jax.ShapeDtypeStruct((TBLP // 128, 128), _f32)),
    )(du_parts, di_parts)


def _scale_body(src_hbm, dst_hbm, et_hbm, cu_hbm, ci_hbm,
                s_out, giu_out, gii_out,
                cu_v, ci_v, a_v, b_v, e_v, s_v, gu_v, gi_v):
    w = _worker_id()
    pltpu.sync_copy(cu_hbm, cu_v)
    pltpu.sync_copy(ci_hbm, ci_v)
    base = w * ECH_W
    nblk = ECH_W // 16
    def blk(b, _):
        off = base + b * 16
        pltpu.sync_copy(src_hbm.at[pl.ds(off, 16)], a_v)
        pltpu.sync_copy(dst_hbm.at[pl.ds(off, 16)], b_v)
        pltpu.sync_copy(et_hbm.at[pl.ds(off, 16)], e_v)

        def row(j, _):
            for g in range(8):
                sl = pl.ds(g * 16, 16)
                av = a_v[j, sl]
                bv = b_v[j, sl]
                ev = e_v[j, sl]
                iu = ev * NUU + av
                ii = ev * NUU + bv
                cuv = plsc.load_gather(cu_v, [iu >> 7, iu & 127])
                civ = plsc.load_gather(ci_v, [ii >> 7, ii & 127])
                s_v[j, sl] = cuv * civ
                gu_v[j, sl] = jnp.minimum(iu, TBL - 1)
                gi_v[j, sl] = jnp.minimum(ii, TBL - 1)
            return 0
        lax.fori_loop(0, 16, row, 0)
        pltpu.sync_copy(s_v, s_out.at[pl.ds(off, 16)])
        pltpu.sync_copy(gu_v, giu_out.at[pl.ds(off, 16)])
        pltpu.sync_copy(gi_v, gii_out.at[pl.ds(off, 16)])
        return 0
    lax.fori_loop(0, nblk, blk, 0)


def _edge_scales(src_all, dst_all, et_all, cu, ci):
    return pl.kernel(
        _scale_body,
        out_type=(jax.ShapeDtypeStruct((ECH, CW), _f32),
                  jax.ShapeDtypeStruct((ECH, CW), _i32),
                  jax.ShapeDtypeStruct((ECH, CW), _i32)),
        mesh=_mesh(),
        compiler_params=_sc_params(),
        scratch_types=[
            pltpu.VMEM((TBLP // 128, 128), _f32),
            pltpu.VMEM((TBLP // 128, 128), _f32),
            pltpu.VMEM((16, CW), _i32),
            pltpu.VMEM((16, CW), _i32),
            pltpu.VMEM((16, CW), _i32),
            pltpu.VMEM((16, CW), _f32),
            pltpu.VMEM((16, CW), _i32),
            pltpu.VMEM((16, CW), _i32),
        ],
    )(src_all, dst_all, et_all, cu, ci)


def _agg_body(mu_hbm, mi_hbm, giu_hbm, gii_hbm, dst_hbm, src_hbm, s_hbm,
              iagg_hbm, uagg_hbm,
              g_v, sc_v, s_v, rows_v, agg_sh, sem):
    c = lax.axis_index("c")
    t = lax.axis_index("s")

    def zrow(i, _):
        for g in range(8):
            rows_v[i, pl.ds(g * 16, 16)] = jnp.zeros((16,), _f32)
        return 0
    lax.fori_loop(0, CW, zrow, 0)
    zbase = t * AGG_T
    def zcp(q, _):
        pltpu.sync_copy(rows_v, agg_sh.at[pl.ds(zbase + q * CW, CW)])
        return 0
    lax.fori_loop(0, AGG_T // CW, zcp, 0)
    rem = AGG_T - (AGG_T // CW) * CW
    pltpu.sync_copy(rows_v.at[pl.ds(0, rem)],
                    agg_sh.at[pl.ds(zbase + (AGG_T // CW) * CW, rem)])
    plsc.subcore_barrier()

    base = t * ECH_T

    def run(tab_hbm, gi_hbm, sx_hbm):
        def blk(b, _):
            off = base + b * K4_BLK
            pltpu.sync_copy(gi_hbm.at[pl.ds(off, K4_BLK)], g_v)
            pltpu.sync_copy(sx_hbm.at[pl.ds(off, K4_BLK)], sc_v)
            pltpu.sync_copy(s_hbm.at[pl.ds(off, K4_BLK)], s_v)

            def chunk(j, _):
                pltpu.async_copy(tab_hbm.at[g_v.at[j]], rows_v, sem).wait()
                def row(i, _):
                    spl = plsc.load_gather(
                        s_v, [jnp.full((16,), j, _i32), jnp.full((16,), i, _i32)])
                    for g in range(8):
                        sl = pl.ds(g * 16, 16)
                        rows_v[i, sl] = rows_v[i, sl] * spl
                    return 0
                lax.fori_loop(0, CW, row, 0)
                pltpu.sync_copy(rows_v, agg_sh.at[sc_v.at[j]], add=True)
                return 0
            lax.fori_loop(0, K4_BLK, chunk, 0)
            return 0
        lax.fori_loop(0, ECH_T // K4_BLK, blk, 0)

    @pl.when(c == 0)
    def _():
        run(mu_hbm, giu_hbm, dst_hbm)
    @pl.when(c == 1)
    def _():
        run(mi_hbm, gii_hbm, src_hbm)
    plsc.subcore_barrier()

    def out(dst_hbm_ref):
        def ocp(q, _):
            sl = pl.ds(zbase + q * CW, CW)
            pltpu.sync_copy(agg_sh.at[sl], rows_v)
            pltpu.sync_copy(rows_v, dst_hbm_ref.at[sl])
            return 0
        lax.fori_loop(0, AGG_T // CW, ocp, 0)
        orem = AGG_T - (AGG_T // CW) * CW
        sl = pl.ds(zbase + (AGG_T // CW) * CW, orem)
        pltpu.sync_copy(agg_sh.at[sl], rows_v.at[pl.ds(0, orem)])
        pltpu.sync_copy(rows_v.at[pl.ds(0, orem)], dst_hbm_ref.at[sl])

    @pl.when(c == 0)
    def _():
        out(iagg_hbm)
    @pl.when(c == 1)
    def _():
        out(uagg_hbm)


def _aggregate(mu, mi, giu, gii, dst_all, src_all, s_all):
    return pl.kernel(
        _agg_body,
        out_type=(jax.ShapeDtypeStruct((AGG_P, DIM), _f32),
                  jax.ShapeDtypeStruct((AGG_P, DIM), _f32)),
        mesh=_mesh(),
        compiler_params=_sc_params(),
        scratch_types=[
            pltpu.VMEM((K4_BLK, CW), _i32),
            pltpu.VMEM((K4_BLK, CW), _i32),
            pltpu.VMEM((K4_BLK, CW), _f32),
            pltpu.VMEM((CW, DIM), _f32),
            pltpu.VMEM_SHARED((AGG_P, DIM), _f32),
            pltpu.SemaphoreType.DMA,
        ],
    )(mu, mi, giu, gii, dst_all, src_all, s_all)


def _fc_body(ua_ref, ia_ref, w_ref, b_ref, uo_ref, io_ref):
    w = w_ref[...]
    b = b_ref[...]
    uh = jnp.maximum(ua_ref[...], 0.0)
    ih = jnp.maximum(ia_ref[...], 0.0)
    z = jnp.zeros((uh.shape[0], DIM - ODIM), _f32)
    uo = jnp.maximum(jnp.dot(uh, w, preferred_element_type=_f32) + b, 0.0)
    io = jnp.maximum(jnp.dot(ih, w, preferred_element_type=_f32) + b, 0.0)
    uo_ref[...] = jnp.concatenate([uo, z], axis=1)
    io_ref[...] = jnp.concatenate([io, z], axis=1)


def _fc(uagg, iagg, fc_w, fc_b2):
    nb = 10
    bs = NUU // nb
    return pl.pallas_call(
        _fc_body,
        grid=(nb,),
        in_specs=[
            pl.BlockSpec((bs, DIM), lambda i: (i, 0)),
            pl.BlockSpec((bs, DIM), lambda i: (i, 0)),
            pl.BlockSpec((DIM, ODIM), lambda i: (0, 0)),
            pl.BlockSpec((1, ODIM), lambda i: (0, 0)),
        ],
        out_specs=[
            pl.BlockSpec((bs, DIM), lambda i: (i, 0)),
            pl.BlockSpec((bs, DIM), lambda i: (i, 0)),
        ],
        out_shape=(jax.ShapeDtypeStruct((NUU, DIM), _f32),
                   jax.ShapeDtypeStruct((NII, DIM), _f32)),
    )(uagg, iagg, fc_w, fc_b2)


def _dec_body(uo_hbm, io_hbm, du_hbm, di_hbm, pred_hbm,
              du_v, di_v, ub_v, ib_v, o_v, sem):
    w = _worker_id()
    pltpu.sync_copy(du_hbm.at[w], du_v)
    pltpu.sync_copy(di_hbm.at[w], di_v)
    ridx = lax.iota(_i32, 16)

    def chunk(j, _):
        pltpu.async_copy(uo_hbm.at[du_v.at[j]], ub_v, sem).wait()
        pltpu.async_copy(io_hbm.at[di_v.at[j]], ib_v, sem).wait()
        def grp(g, _):
            rows = ridx + g * 16
            acc = jnp.zeros((16,), _f32)
            for d in range(ODIM):
                col = jnp.full((16,), d, _i32)
                uv = plsc.load_gather(ub_v, [rows, col])
                iv = plsc.load_gather(ib_v, [rows, col])
                acc = acc + uv * iv
            o_v[pl.ds(g * 16, 16)] = acc
            return 0
        lax.fori_loop(0, 8, grp, 0)
        pltpu.sync_copy(o_v, pred_hbm.at[w].at[j])
        return 0
    lax.fori_loop(0, DCH_W, chunk, 0)


def _decode(uo, io, du_all, di_all):
    return pl.kernel(
        _dec_body,
        out_type=jax.ShapeDtypeStruct((NW, DCH_W, CW), _f32),
        mesh=_mesh(),
        compiler_params=_sc_params(),
        scratch_types=[
            pltpu.VMEM((DCH_W, CW), _i32),
            pltpu.VMEM((DCH_W, CW), _i32),
            pltpu.VMEM((CW, DIM), _f32),
            pltpu.VMEM((CW, DIM), _f32),
            pltpu.VMEM((CW,), _f32),
            pltpu.SemaphoreType.DMA,
        ],
    )(uo, io, du_all, di_all)


def kernel(ufeat, ifeat, W, fc_w, fc_b, enc_edge_index, enc_etypes, dec_edge_index):
    src = enc_edge_index[0].astype(_i32)
    dst = enc_edge_index[1].astype(_i32)
    et = enc_etypes.astype(_i32)
    npad = EP - EE
    src_all = jnp.concatenate([src, jnp.full((npad,), NUU, _i32)]).reshape(ECH, CW)
    dst_all = jnp.concatenate([dst, jnp.full((npad,), NII, _i32)]).reshape(ECH, CW)
    et_all = jnp.concatenate([et, jnp.full((npad,), RR - 1, _i32)]).reshape(ECH, CW)
    dpad = EDP - EDD
    du_all = jnp.concatenate(
        [dec_edge_index[0].astype(_i32), jnp.zeros((dpad,), _i32)]
    ).reshape(NW, DCH_W, CW)
    di_all = jnp.concatenate(
        [dec_edge_index[1].astype(_i32), jnp.zeros((dpad,), _i32)]
    ).reshape(NW, DCH_W, CW)

    du_parts, di_parts = _degrees(src_all, dst_all, et_all)
    mu, mi = _projections(ufeat, ifeat, W)
    cu, ci = _norms(du_parts, di_parts)
    s_all, giu, gii = _edge_scales(src_all, dst_all, et_all, cu, ci)
    iagg_p, uagg_p = _aggregate(mu, mi, giu, gii, dst_all, src_all, s_all)
    iagg, uagg = iagg_p[:NII], uagg_p[:NUU]
    uo, io = _fc(uagg, iagg, fc_w, fc_b.reshape(1, ODIM))
    pred = _decode(uo, io, du_all, di_all)
    return pred.reshape(EDP)[:EDD][:, None]


# speedup vs baseline: 26.1050x; 1.1628x over previous
import jax
import jax.numpy as jnp
from jax import lax
from jax.experimental import pallas as pl
from jax.experimental.pallas import tpu as pltpu
from jax.experimental.pallas import tpu_sc as plsc

NUU = 10000
NII = 10000
EE = 320000
EDD = 100000
DIM = 128
ODIM = 64
RR = 5

NC = 2
NS = 16
NW = NC * NS
CW = 128

TBL = RR * NUU
TBLP = TBL + 48
EP = 327680
ECH = EP // CW
ECH_T = ECH // NS
ECH_W = ECH // NW
EDP = 102400
DCH = EDP // CW
DCH_W = DCH // NW

AGG_P = NUU + 112
AGG_T = AGG_P // NS
K4_BLK = 16

_i32 = jnp.int32
_f32 = jnp.float32


def _mesh():
    return plsc.VectorSubcoreMesh(core_axis_name="c", subcore_axis_name="s")


def _sc_params():
    return pltpu.CompilerParams(needs_layout_passes=False)


def _worker_id():
    return lax.axis_index("s") * NC + lax.axis_index("c")


def _deg_body(src_hbm, dst_hbm, et_hbm, degu_hbm, degi_hbm,
              a_v, e_v, deg_v):
    c = lax.axis_index("c")
    t = lax.axis_index("s")

    def zrow(q, _):
        for g in range(8):
            deg_v[q, pl.ds(g * 16, 16)] = jnp.zeros((16,), _f32)
        return 0
    lax.fori_loop(0, TBLP // 128, zrow, 0)

    base = t * ECH_T

    def run(node_hbm):
        def blk(b, _):
            off = base + b * K4_BLK
            pltpu.sync_copy(node_hbm.at[pl.ds(off, K4_BLK)], a_v)
            pltpu.sync_copy(et_hbm.at[pl.ds(off, K4_BLK)], e_v)
            def row(j, _):
                for g in range(8):
                    sl = pl.ds(g * 16, 16)
                    idx = e_v[j, sl] * NUU + a_v[j, sl]
                    plsc.addupdate_scatter(
                        deg_v, [idx >> 7, idx & 127],
                        jnp.full((16,), 1.0, _f32))
                return 0
            lax.fori_loop(0, K4_BLK, row, 0)
            return 0
        lax.fori_loop(0, ECH_T // K4_BLK, blk, 0)

    @pl.when(c == 0)
    def _():
        run(src_hbm)
        pltpu.sync_copy(deg_v, degu_hbm.at[t])
    @pl.when(c == 1)
    def _():
        run(dst_hbm)
        pltpu.sync_copy(deg_v, degi_hbm.at[t])


def _degrees(src_all, dst_all, et_all):
    return pl.kernel(
        _deg_body,
        out_type=(jax.ShapeDtypeStruct((NS, TBLP // 128, 128), _f32),
                  jax.ShapeDtypeStruct((NS, TBLP // 128, 128), _f32)),
        mesh=_mesh(),
        compiler_params=_sc_params(),
        scratch_types=[
            pltpu.VMEM((K4_BLK, CW), _i32),
            pltpu.VMEM((K4_BLK, CW), _i32),
            pltpu.VMEM((TBLP // 128, 128), _f32),
        ],
    )(src_all, dst_all, et_all)


def _mm_body(u_ref, i_ref, w_ref, mu_ref, mi_ref):
    w = w_ref[0]
    mu_ref[...] = jnp.dot(u_ref[...], w, preferred_element_type=_f32)
    mi_ref[...] = jnp.dot(i_ref[...], w, preferred_element_type=_f32)


def _projections(ufeat, ifeat, W):
    nb = 10
    bs = NUU // nb
    return pl.pallas_call(
        _mm_body,
        grid=(RR, nb),
        in_specs=[
            pl.BlockSpec((bs, DIM), lambda r, i: (i, 0)),
            pl.BlockSpec((bs, DIM), lambda r, i: (i, 0)),
            pl.BlockSpec((1, DIM, DIM), lambda r, i: (r, 0, 0)),
        ],
        out_specs=[
            pl.BlockSpec((bs, DIM), lambda r, i: (r * 10 + i, 0)),
            pl.BlockSpec((bs, DIM), lambda r, i: (r * 10 + i, 0)),
        ],
        out_shape=(jax.ShapeDtypeStruct((TBL, DIM), _f32),
                   jax.ShapeDtypeStruct((TBL, DIM), _f32)),
    )(ufeat, ifeat, W)


def _norm_body(du_ref, di_ref, cu_ref, ci_ref):
    du = jnp.sum(du_ref[...], axis=0)
    di = jnp.sum(di_ref[...], axis=0)
    cu_ref[...] = lax.rsqrt(jnp.maximum(du, 1.0))
    ci_ref[...] = lax.rsqrt(jnp.maximum(di, 1.0))


def _norms(du_parts, di_parts):
    return pl.pallas_call(
        _norm_body,
        out_shape=(jax.ShapeDtypeStruct((TBLP // 128, 128), _f32),
                   jax.ShapeDtypeStruct((TBLP // 128, 128), _f32)),
    )(du_parts, di_parts)


def _scale_body(src_hbm, dst_hbm, et_hbm, cu_hbm, ci_hbm,
                s_out, giu_out, gii_out,
                cu_v, ci_v, a_v, b_v, e_v, s_v, gu_v, gi_v):
    w = _worker_id()
    pltpu.sync_copy(cu_hbm, cu_v)
    pltpu.sync_copy(ci_hbm, ci_v)
    base = w * ECH_W
    nblk = ECH_W // 16
    def blk(b, _):
        off = base + b * 16
        pltpu.sync_copy(src_hbm.at[pl.ds(off, 16)], a_v)
        pltpu.sync_copy(dst_hbm.at[pl.ds(off, 16)], b_v)
        pltpu.sync_copy(et_hbm.at[pl.ds(off, 16)], e_v)

        def row(j, _):
            for g in range(8):
                sl = pl.ds(g * 16, 16)
                av = a_v[j, sl]
                bv = b_v[j, sl]
                ev = e_v[j, sl]
                iu = ev * NUU + av
                ii = ev * NUU + bv
                cuv = plsc.load_gather(cu_v, [iu >> 7, iu & 127])
                civ = plsc.load_gather(ci_v, [ii >> 7, ii & 127])
                s_v[j, sl] = cuv * civ
                gu_v[j, sl] = jnp.minimum(iu, TBL - 1)
                gi_v[j, sl] = jnp.minimum(ii, TBL - 1)
            return 0
        lax.fori_loop(0, 16, row, 0)
        pltpu.sync_copy(s_v, s_out.at[pl.ds(off, 16)])
        pltpu.sync_copy(gu_v, giu_out.at[pl.ds(off, 16)])
        pltpu.sync_copy(gi_v, gii_out.at[pl.ds(off, 16)])
        return 0
    lax.fori_loop(0, nblk, blk, 0)


def _edge_scales(src_all, dst_all, et_all, cu, ci):
    return pl.kernel(
        _scale_body,
        out_type=(jax.ShapeDtypeStruct((ECH, CW), _f32),
                  jax.ShapeDtypeStruct((ECH, CW), _i32),
                  jax.ShapeDtypeStruct((ECH, CW), _i32)),
        mesh=_mesh(),
        compiler_params=_sc_params(),
        scratch_types=[
            pltpu.VMEM((TBLP // 128, 128), _f32),
            pltpu.VMEM((TBLP // 128, 128), _f32),
            pltpu.VMEM((16, CW), _i32),
            pltpu.VMEM((16, CW), _i32),
            pltpu.VMEM((16, CW), _i32),
            pltpu.VMEM((16, CW), _f32),
            pltpu.VMEM((16, CW), _i32),
            pltpu.VMEM((16, CW), _i32),
        ],
    )(src_all, dst_all, et_all, cu, ci)


def _agg_body(mu_hbm, mi_hbm, giu_hbm, gii_hbm, dst_hbm, src_hbm, s_hbm,
              iagg_hbm, uagg_hbm,
              g_v, sc_v, s_v, rows_a, rows_b, agg_sh, sem_a, sem_b):
    c = lax.axis_index("c")
    t = lax.axis_index("s")

    def zrow(i, _):
        for g in range(8):
            rows_a[i, pl.ds(g * 16, 16)] = jnp.zeros((16,), _f32)
        return 0
    lax.fori_loop(0, CW, zrow, 0)
    zbase = t * AGG_T
    def zcp(q, _):
        pltpu.sync_copy(rows_a, agg_sh.at[pl.ds(zbase + q * CW, CW)])
        return 0
    lax.fori_loop(0, AGG_T // CW, zcp, 0)
    rem = AGG_T - (AGG_T // CW) * CW
    pltpu.sync_copy(rows_a.at[pl.ds(0, rem)],
                    agg_sh.at[pl.ds(zbase + (AGG_T // CW) * CW, rem)])
    plsc.subcore_barrier()

    base = t * ECH_T

    def run(tab_hbm, gi_hbm, sx_hbm):
        def scale_scatter(buf, j):
            def row(i, _):
                spl = plsc.load_gather(
                    s_v, [jnp.full((16,), j, _i32), jnp.full((16,), i, _i32)])
                for g in range(8):
                    sl = pl.ds(g * 16, 16)
                    buf[i, sl] = buf[i, sl] * spl
                return 0
            lax.fori_loop(0, CW, row, 0)
            pltpu.sync_copy(buf, agg_sh.at[sc_v.at[j]], add=True)

        def blk(b, _):
            off = base + b * K4_BLK
            pltpu.sync_copy(gi_hbm.at[pl.ds(off, K4_BLK)], g_v)
            pltpu.sync_copy(sx_hbm.at[pl.ds(off, K4_BLK)], sc_v)
            pltpu.sync_copy(s_hbm.at[pl.ds(off, K4_BLK)], s_v)

            pltpu.async_copy(tab_hbm.at[g_v.at[0]], rows_a, sem_a)

            def pair(q, _):
                j0 = 2 * q
                j1 = j0 + 1
                pltpu.async_copy(tab_hbm.at[g_v.at[j1]], rows_b, sem_b)
                pltpu.make_async_copy(
                    tab_hbm.at[g_v.at[j0]], rows_a, sem_a).wait()
                scale_scatter(rows_a, j0)
                @pl.when(q < K4_BLK // 2 - 1)
                def _():
                    pltpu.async_copy(
                        tab_hbm.at[g_v.at[j0 + 2]], rows_a, sem_a)
                pltpu.make_async_copy(
                    tab_hbm.at[g_v.at[j1]], rows_b, sem_b).wait()
                scale_scatter(rows_b, j1)
                return 0
            lax.fori_loop(0, K4_BLK // 2, pair, 0)
            return 0
        lax.fori_loop(0, ECH_T // K4_BLK, blk, 0)

    @pl.when(c == 0)
    def _():
        run(mu_hbm, giu_hbm, dst_hbm)
    @pl.when(c == 1)
    def _():
        run(mi_hbm, gii_hbm, src_hbm)
    plsc.subcore_barrier()

    def out(dst_hbm_ref):
        def ocp(q, _):
            sl = pl.ds(zbase + q * CW, CW)
            pltpu.sync_copy(agg_sh.at[sl], rows_a)
            pltpu.sync_copy(rows_a, dst_hbm_ref.at[sl])
            return 0
        lax.fori_loop(0, AGG_T // CW, ocp, 0)
        orem = AGG_T - (AGG_T // CW) * CW
        sl = pl.ds(zbase + (AGG_T // CW) * CW, orem)
        pltpu.sync_copy(agg_sh.at[sl], rows_a.at[pl.ds(0, orem)])
        pltpu.sync_copy(rows_a.at[pl.ds(0, orem)], dst_hbm_ref.at[sl])

    @pl.when(c == 0)
    def _():
        out(iagg_hbm)
    @pl.when(c == 1)
    def _():
        out(uagg_hbm)


def _aggregate(mu, mi, giu, gii, dst_all, src_all, s_all):
    return pl.kernel(
        _agg_body,
        out_type=(jax.ShapeDtypeStruct((AGG_P, DIM), _f32),
                  jax.ShapeDtypeStruct((AGG_P, DIM), _f32)),
        mesh=_mesh(),
        compiler_params=_sc_params(),
        scratch_types=[
            pltpu.VMEM((K4_BLK, CW), _i32),
            pltpu.VMEM((K4_BLK, CW), _i32),
            pltpu.VMEM((K4_BLK, CW), _f32),
            pltpu.VMEM((CW, DIM), _f32),
            pltpu.VMEM((CW, DIM), _f32),
            pltpu.VMEM_SHARED((AGG_P, DIM), _f32),
            pltpu.SemaphoreType.DMA,
            pltpu.SemaphoreType.DMA,
        ],
    )(mu, mi, giu, gii, dst_all, src_all, s_all)


def _fc_body(ua_ref, ia_ref, w_ref, b_ref, uo_ref, io_ref):
    w = w_ref[...]
    b = b_ref[...]
    uh = jnp.maximum(ua_ref[...], 0.0)
    ih = jnp.maximum(ia_ref[...], 0.0)
    z = jnp.zeros((uh.shape[0], DIM - ODIM), _f32)
    uo = jnp.maximum(jnp.dot(uh, w, preferred_element_type=_f32) + b, 0.0)
    io = jnp.maximum(jnp.dot(ih, w, preferred_element_type=_f32) + b, 0.0)
    uo_ref[...] = jnp.concatenate([uo, z], axis=1)
    io_ref[...] = jnp.concatenate([io, z], axis=1)


def _fc(uagg, iagg, fc_w, fc_b2):
    nb = 10
    bs = NUU // nb
    return pl.pallas_call(
        _fc_body,
        grid=(nb,),
        in_specs=[
            pl.BlockSpec((bs, DIM), lambda i: (i, 0)),
            pl.BlockSpec((bs, DIM), lambda i: (i, 0)),
            pl.BlockSpec((DIM, ODIM), lambda i: (0, 0)),
            pl.BlockSpec((1, ODIM), lambda i: (0, 0)),
        ],
        out_specs=[
            pl.BlockSpec((bs, DIM), lambda i: (i, 0)),
            pl.BlockSpec((bs, DIM), lambda i: (i, 0)),
        ],
        out_shape=(jax.ShapeDtypeStruct((NUU, DIM), _f32),
                   jax.ShapeDtypeStruct((NII, DIM), _f32)),
    )(uagg, iagg, fc_w, fc_b2)


def _dec_body(uo_hbm, io_hbm, du_hbm, di_hbm, pred_hbm,
              du_v, di_v, ub_v, ib_v, o_v, sem):
    w = _worker_id()
    pltpu.sync_copy(du_hbm.at[w], du_v)
    pltpu.sync_copy(di_hbm.at[w], di_v)
    ridx = lax.iota(_i32, 16)

    def chunk(j, _):
        pltpu.async_copy(uo_hbm.at[du_v.at[j]], ub_v, sem).wait()
        pltpu.async_copy(io_hbm.at[di_v.at[j]], ib_v, sem).wait()
        def grp(g, _):
            rows = ridx + g * 16
            acc = jnp.zeros((16,), _f32)
            for d in range(ODIM):
                col = jnp.full((16,), d, _i32)
                uv = plsc.load_gather(ub_v, [rows, col])
                iv = plsc.load_gather(ib_v, [rows, col])
                acc = acc + uv * iv
            o_v[pl.ds(g * 16, 16)] = acc
            return 0
        lax.fori_loop(0, 8, grp, 0)
        pltpu.sync_copy(o_v, pred_hbm.at[w].at[j])
        return 0
    lax.fori_loop(0, DCH_W, chunk, 0)


def _decode(uo, io, du_all, di_all):
    return pl.kernel(
        _dec_body,
        out_type=jax.ShapeDtypeStruct((NW, DCH_W, CW), _f32),
        mesh=_mesh(),
        compiler_params=_sc_params(),
        scratch_types=[
            pltpu.VMEM((DCH_W, CW), _i32),
            pltpu.VMEM((DCH_W, CW), _i32),
            pltpu.VMEM((CW, DIM), _f32),
            pltpu.VMEM((CW, DIM), _f32),
            pltpu.VMEM((CW,), _f32),
            pltpu.SemaphoreType.DMA,
        ],
    )(uo, io, du_all, di_all)


def kernel(ufeat, ifeat, W, fc_w, fc_b, enc_edge_index, enc_etypes, dec_edge_index):
    src = enc_edge_index[0].astype(_i32)
    dst = enc_edge_index[1].astype(_i32)
    et = enc_etypes.astype(_i32)
    npad = EP - EE
    src_all = jnp.concatenate([src, jnp.full((npad,), NUU, _i32)]).reshape(ECH, CW)
    dst_all = jnp.concatenate([dst, jnp.full((npad,), NII, _i32)]).reshape(ECH, CW)
    et_all = jnp.concatenate([et, jnp.full((npad,), RR - 1, _i32)]).reshape(ECH, CW)
    dpad = EDP - EDD
    du_all = jnp.concatenate(
        [dec_edge_index[0].astype(_i32), jnp.zeros((dpad,), _i32)]
    ).reshape(NW, DCH_W, CW)
    di_all = jnp.concatenate(
        [dec_edge_index[1].astype(_i32), jnp.zeros((dpad,), _i32)]
    ).reshape(NW, DCH_W, CW)

    du_parts, di_parts = _degrees(src_all, dst_all, et_all)
    mu, mi = _projections(ufeat, ifeat, W)
    cu, ci = _norms(du_parts, di_parts)
    s_all, giu, gii = _edge_scales(src_all, dst_all, et_all, cu, ci)
    iagg_p, uagg_p = _aggregate(mu, mi, giu, gii, dst_all, src_all, s_all)
    iagg, uagg = iagg_p[:NII], uagg_p[:NUU]
    uo, io = _fc(uagg, iagg, fc_w, fc_b.reshape(1, ODIM))
    pred = _decode(uo, io, du_all, di_all)
    return pred.reshape(EDP)[:EDD][:, None]


# speedup vs baseline: 26.3091x; 1.0078x over previous
import jax
import jax.numpy as jnp
from jax import lax
from jax.experimental import pallas as pl
from jax.experimental.pallas import tpu as pltpu
from jax.experimental.pallas import tpu_sc as plsc

NUU = 10000
NII = 10000
EE = 320000
EDD = 100000
DIM = 128
ODIM = 64
RR = 5

NC = 2
NS = 16
NW = NC * NS
CW = 128

TBL = RR * NUU
TBLP = TBL + 48
EP = 327680
ECH = EP // CW
ECH_T = ECH // NS
ECH_W = ECH // NW
EDP = 102400
DCH = EDP // CW
DCH_W = DCH // NW

AGG_P = NUU + 112
AGG_T = AGG_P // NS
K4_BLK = 16

_i32 = jnp.int32
_f32 = jnp.float32


def _mesh():
    return plsc.VectorSubcoreMesh(core_axis_name="c", subcore_axis_name="s")


def _sc_params():
    return pltpu.CompilerParams(needs_layout_passes=False)


def _worker_id():
    return lax.axis_index("s") * NC + lax.axis_index("c")


def _deg_body(src_hbm, dst_hbm, et_hbm, degu_hbm, degi_hbm,
              a_v, e_v, deg_v):
    c = lax.axis_index("c")
    t = lax.axis_index("s")

    def zrow(q, _):
        for g in range(8):
            deg_v[q, pl.ds(g * 16, 16)] = jnp.zeros((16,), _f32)
        return 0
    lax.fori_loop(0, TBLP // 128, zrow, 0)

    base = t * ECH_T

    def run(node_hbm):
        def blk(b, _):
            off = base + b * K4_BLK
            pltpu.sync_copy(node_hbm.at[pl.ds(off, K4_BLK)], a_v)
            pltpu.sync_copy(et_hbm.at[pl.ds(off, K4_BLK)], e_v)
            def row(j, _):
                for g in range(8):
                    sl = pl.ds(g * 16, 16)
                    idx = e_v[j, sl] * NUU + a_v[j, sl]
                    plsc.addupdate_scatter(
                        deg_v, [idx >> 7, idx & 127],
                        jnp.full((16,), 1.0, _f32))
                return 0
            lax.fori_loop(0, K4_BLK, row, 0)
            return 0
        lax.fori_loop(0, ECH_T // K4_BLK, blk, 0)

    @pl.when(c == 0)
    def _():
        run(src_hbm)
        pltpu.sync_copy(deg_v, degu_hbm.at[t])
    @pl.when(c == 1)
    def _():
        run(dst_hbm)
        pltpu.sync_copy(deg_v, degi_hbm.at[t])


def _degrees(src_all, dst_all, et_all):
    return pl.kernel(
        _deg_body,
        out_type=(jax.ShapeDtypeStruct((NS, TBLP // 128, 128), _f32),
                  jax.ShapeDtypeStruct((NS, TBLP // 128, 128), _f32)),
        mesh=_mesh(),
        compiler_params=_sc_params(),
        scratch_types=[
            pltpu.VMEM((K4_BLK, CW), _i32),
            pltpu.VMEM((K4_BLK, CW), _i32),
            pltpu.VMEM((TBLP // 128, 128), _f32),
        ],
    )(src_all, dst_all, et_all)


def _mm_body(u_ref, i_ref, w_ref, mu_ref, mi_ref):
    w = w_ref[0]
    mu_ref[...] = jnp.dot(u_ref[...], w, preferred_element_type=_f32)
    mi_ref[...] = jnp.dot(i_ref[...], w, preferred_element_type=_f32)


def _projections(ufeat, ifeat, W):
    nb = 10
    bs = NUU // nb
    return pl.pallas_call(
        _mm_body,
        grid=(RR, nb),
        in_specs=[
            pl.BlockSpec((bs, DIM), lambda r, i: (i, 0)),
            pl.BlockSpec((bs, DIM), lambda r, i: (i, 0)),
            pl.BlockSpec((1, DIM, DIM), lambda r, i: (r, 0, 0)),
        ],
        out_specs=[
            pl.BlockSpec((bs, DIM), lambda r, i: (r * 10 + i, 0)),
            pl.BlockSpec((bs, DIM), lambda r, i: (r * 10 + i, 0)),
        ],
        out_shape=(jax.ShapeDtypeStruct((TBL, DIM), _f32),
                   jax.ShapeDtypeStruct((TBL, DIM), _f32)),
    )(ufeat, ifeat, W)


def _norm_body(du_ref, di_ref, cu_ref, ci_ref):
    du = jnp.sum(du_ref[...], axis=0)
    di = jnp.sum(di_ref[...], axis=0)
    cu_ref[...] = lax.rsqrt(jnp.maximum(du, 1.0))
    ci_ref[...] = lax.rsqrt(jnp.maximum(di, 1.0))


def _norms(du_parts, di_parts):
    return pl.pallas_call(
        _norm_body,
        out_shape=(jax.ShapeDtypeStruct((TBLP // 128, 128), _f32),
                   jax.ShapeDtypeStruct((TBLP // 128, 128), _f32)),
    )(du_parts, di_parts)


def _scale_body(src_hbm, dst_hbm, et_hbm, cu_hbm, ci_hbm,
                s_out, giu_out, gii_out,
                cu_v, ci_v, a_v, b_v, e_v, s_v, gu_v, gi_v):
    w = _worker_id()
    pltpu.sync_copy(cu_hbm, cu_v)
    pltpu.sync_copy(ci_hbm, ci_v)
    base = w * ECH_W
    nblk = ECH_W // 16
    def blk(b, _):
        off = base + b * 16
        pltpu.sync_copy(src_hbm.at[pl.ds(off, 16)], a_v)
        pltpu.sync_copy(dst_hbm.at[pl.ds(off, 16)], b_v)
        pltpu.sync_copy(et_hbm.at[pl.ds(off, 16)], e_v)

        def row(j, _):
            for g in range(8):
                sl = pl.ds(g * 16, 16)
                av = a_v[j, sl]
                bv = b_v[j, sl]
                ev = e_v[j, sl]
                iu = ev * NUU + av
                ii = ev * NUU + bv
                cuv = plsc.load_gather(cu_v, [iu >> 7, iu & 127])
                civ = plsc.load_gather(ci_v, [ii >> 7, ii & 127])
                s_v[j, sl] = cuv * civ
                gu_v[j, sl] = jnp.minimum(iu, TBL - 1)
                gi_v[j, sl] = jnp.minimum(ii, TBL - 1)
            return 0
        lax.fori_loop(0, 16, row, 0)
        pltpu.sync_copy(s_v, s_out.at[pl.ds(off, 16)])
        pltpu.sync_copy(gu_v, giu_out.at[pl.ds(off, 16)])
        pltpu.sync_copy(gi_v, gii_out.at[pl.ds(off, 16)])
        return 0
    lax.fori_loop(0, nblk, blk, 0)


def _edge_scales(src_all, dst_all, et_all, cu, ci):
    return pl.kernel(
        _scale_body,
        out_type=(jax.ShapeDtypeStruct((ECH, CW), _f32),
                  jax.ShapeDtypeStruct((ECH, CW), _i32),
                  jax.ShapeDtypeStruct((ECH, CW), _i32)),
        mesh=_mesh(),
        compiler_params=_sc_params(),
        scratch_types=[
            pltpu.VMEM((TBLP // 128, 128), _f32),
            pltpu.VMEM((TBLP // 128, 128), _f32),
            pltpu.VMEM((16, CW), _i32),
            pltpu.VMEM((16, CW), _i32),
            pltpu.VMEM((16, CW), _i32),
            pltpu.VMEM((16, CW), _f32),
            pltpu.VMEM((16, CW), _i32),
            pltpu.VMEM((16, CW), _i32),
        ],
    )(src_all, dst_all, et_all, cu, ci)


def _agg_body(mu_hbm, mi_hbm, giu_hbm, gii_hbm, dst_hbm, src_hbm, s_hbm,
              iagg_hbm, uagg_hbm,
              g_v, sc_v, s_v, rows_a, rows_b, agg_sh, sem_a, sem_b):
    c = lax.axis_index("c")
    t = lax.axis_index("s")

    def zrow(i, _):
        for g in range(8):
            rows_a[i, pl.ds(g * 16, 16)] = jnp.zeros((16,), _f32)
        return 0
    lax.fori_loop(0, CW, zrow, 0)
    zbase = t * AGG_T
    def zcp(q, _):
        pltpu.sync_copy(rows_a, agg_sh.at[pl.ds(zbase + q * CW, CW)])
        return 0
    lax.fori_loop(0, AGG_T // CW, zcp, 0)
    rem = AGG_T - (AGG_T // CW) * CW
    pltpu.sync_copy(rows_a.at[pl.ds(0, rem)],
                    agg_sh.at[pl.ds(zbase + (AGG_T // CW) * CW, rem)])
    plsc.subcore_barrier()

    base = t * ECH_T

    def run(tab_hbm, gi_hbm, sx_hbm):
        def scale_scatter(buf, j):
            jv = jnp.full((16,), j, _i32)
            def row(q, _):
                i0 = q * 4
                for u in range(4):
                    spl = plsc.load_gather(
                        s_v, [jv, jnp.full((16,), i0 + u, _i32)])
                    for g in range(8):
                        sl = pl.ds(g * 16, 16)
                        buf[i0 + u, sl] = buf[i0 + u, sl] * spl
                return 0
            lax.fori_loop(0, CW // 4, row, 0)
            pltpu.sync_copy(buf, agg_sh.at[sc_v.at[j]], add=True)

        def blk(b, _):
            off = base + b * K4_BLK
            pltpu.sync_copy(gi_hbm.at[pl.ds(off, K4_BLK)], g_v)
            pltpu.sync_copy(sx_hbm.at[pl.ds(off, K4_BLK)], sc_v)
            pltpu.sync_copy(s_hbm.at[pl.ds(off, K4_BLK)], s_v)

            pltpu.async_copy(tab_hbm.at[g_v.at[0]], rows_a, sem_a)

            def pair(q, _):
                j0 = 2 * q
                j1 = j0 + 1
                pltpu.async_copy(tab_hbm.at[g_v.at[j1]], rows_b, sem_b)
                pltpu.make_async_copy(
                    tab_hbm.at[g_v.at[j0]], rows_a, sem_a).wait()
                scale_scatter(rows_a, j0)
                @pl.when(q < K4_BLK // 2 - 1)
                def _():
                    pltpu.async_copy(
                        tab_hbm.at[g_v.at[j0 + 2]], rows_a, sem_a)
                pltpu.make_async_copy(
                    tab_hbm.at[g_v.at[j1]], rows_b, sem_b).wait()
                scale_scatter(rows_b, j1)
                return 0
            lax.fori_loop(0, K4_BLK // 2, pair, 0)
            return 0
        lax.fori_loop(0, ECH_T // K4_BLK, blk, 0)

    @pl.when(c == 0)
    def _():
        run(mu_hbm, giu_hbm, dst_hbm)
    @pl.when(c == 1)
    def _():
        run(mi_hbm, gii_hbm, src_hbm)
    plsc.subcore_barrier()

    def out(dst_hbm_ref):
        def ocp(q, _):
            sl = pl.ds(zbase + q * CW, CW)
            pltpu.sync_copy(agg_sh.at[sl], rows_a)
            pltpu.sync_copy(rows_a, dst_hbm_ref.at[sl])
            return 0
        lax.fori_loop(0, AGG_T // CW, ocp, 0)
        orem = AGG_T - (AGG_T // CW) * CW
        sl = pl.ds(zbase + (AGG_T // CW) * CW, orem)
        pltpu.sync_copy(agg_sh.at[sl], rows_a.at[pl.ds(0, orem)])
        pltpu.sync_copy(rows_a.at[pl.ds(0, orem)], dst_hbm_ref.at[sl])

    @pl.when(c == 0)
    def _():
        out(iagg_hbm)
    @pl.when(c == 1)
    def _():
        out(uagg_hbm)


def _aggregate(mu, mi, giu, gii, dst_all, src_all, s_all):
    return pl.kernel(
        _agg_body,
        out_type=(jax.ShapeDtypeStruct((AGG_P, DIM), _f32),
                  jax.ShapeDtypeStruct((AGG_P, DIM), _f32)),
        mesh=_mesh(),
        compiler_params=_sc_params(),
        scratch_types=[
            pltpu.VMEM((K4_BLK, CW), _i32),
            pltpu.VMEM((K4_BLK, CW), _i32),
            pltpu.VMEM((K4_BLK, CW), _f32),
            pltpu.VMEM((CW, DIM), _f32),
            pltpu.VMEM((CW, DIM), _f32),
            pltpu.VMEM_SHARED((AGG_P, DIM), _f32),
            pltpu.SemaphoreType.DMA,
            pltpu.SemaphoreType.DMA,
        ],
    )(mu, mi, giu, gii, dst_all, src_all, s_all)


def _fc_body(ua_ref, ia_ref, w_ref, b_ref, uo_ref, io_ref):
    w = w_ref[...]
    b = b_ref[...]
    uh = jnp.maximum(ua_ref[...], 0.0)
    ih = jnp.maximum(ia_ref[...], 0.0)
    z = jnp.zeros((uh.shape[0], DIM - ODIM), _f32)
    uo = jnp.maximum(jnp.dot(uh, w, preferred_element_type=_f32) + b, 0.0)
    io = jnp.maximum(jnp.dot(ih, w, preferred_element_type=_f32) + b, 0.0)
    uo_ref[...] = jnp.concatenate([uo, z], axis=1)
    io_ref[...] = jnp.concatenate([io, z], axis=1)


def _fc(uagg, iagg, fc_w, fc_b2):
    nb = 10
    bs = NUU // nb
    return pl.pallas_call(
        _fc_body,
        grid=(nb,),
        in_specs=[
            pl.BlockSpec((bs, DIM), lambda i: (i, 0)),
            pl.BlockSpec((bs, DIM), lambda i: (i, 0)),
            pl.BlockSpec((DIM, ODIM), lambda i: (0, 0)),
            pl.BlockSpec((1, ODIM), lambda i: (0, 0)),
        ],
        out_specs=[
            pl.BlockSpec((bs, DIM), lambda i: (i, 0)),
            pl.BlockSpec((bs, DIM), lambda i: (i, 0)),
        ],
        out_shape=(jax.ShapeDtypeStruct((NUU, DIM), _f32),
                   jax.ShapeDtypeStruct((NII, DIM), _f32)),
    )(uagg, iagg, fc_w, fc_b2)


def _dec_body(uo_hbm, io_hbm, du_hbm, di_hbm, pred_hbm,
              du_v, di_v, ub_v, ib_v, o_v, sem):
    w = _worker_id()
    pltpu.sync_copy(du_hbm.at[w], du_v)
    pltpu.sync_copy(di_hbm.at[w], di_v)
    ridx = lax.iota(_i32, 16)

    def chunk(j, _):
        pltpu.async_copy(uo_hbm.at[du_v.at[j]], ub_v, sem).wait()
        pltpu.async_copy(io_hbm.at[di_v.at[j]], ib_v, sem).wait()
        def grp(g, _):
            rows = ridx + g * 16
            acc = jnp.zeros((16,), _f32)
            for d in range(ODIM):
                col = jnp.full((16,), d, _i32)
                uv = plsc.load_gather(ub_v, [rows, col])
                iv = plsc.load_gather(ib_v, [rows, col])
                acc = acc + uv * iv
            o_v[pl.ds(g * 16, 16)] = acc
            return 0
        lax.fori_loop(0, 8, grp, 0)
        pltpu.sync_copy(o_v, pred_hbm.at[w].at[j])
        return 0
    lax.fori_loop(0, DCH_W, chunk, 0)


def _decode(uo, io, du_all, di_all):
    return pl.kernel(
        _dec_body,
        out_type=jax.ShapeDtypeStruct((NW, DCH_W, CW), _f32),
        mesh=_mesh(),
        compiler_params=_sc_params(),
        scratch_types=[
            pltpu.VMEM((DCH_W, CW), _i32),
            pltpu.VMEM((DCH_W, CW), _i32),
            pltpu.VMEM((CW, DIM), _f32),
            pltpu.VMEM((CW, DIM), _f32),
            pltpu.VMEM((CW,), _f32),
            pltpu.SemaphoreType.DMA,
        ],
    )(uo, io, du_all, di_all)


def kernel(ufeat, ifeat, W, fc_w, fc_b, enc_edge_index, enc_etypes, dec_edge_index):
    src = enc_edge_index[0].astype(_i32)
    dst = enc_edge_index[1].astype(_i32)
    et = enc_etypes.astype(_i32)
    npad = EP - EE
    src_all = jnp.concatenate([src, jnp.full((npad,), NUU, _i32)]).reshape(ECH, CW)
    dst_all = jnp.concatenate([dst, jnp.full((npad,), NII, _i32)]).reshape(ECH, CW)
    et_all = jnp.concatenate([et, jnp.full((npad,), RR - 1, _i32)]).reshape(ECH, CW)
    dpad = EDP - EDD
    du_all = jnp.concatenate(
        [dec_edge_index[0].astype(_i32), jnp.zeros((dpad,), _i32)]
    ).reshape(NW, DCH_W, CW)
    di_all = jnp.concatenate(
        [dec_edge_index[1].astype(_i32), jnp.zeros((dpad,), _i32)]
    ).reshape(NW, DCH_W, CW)

    du_parts, di_parts = _degrees(src_all, dst_all, et_all)
    mu, mi = _projections(ufeat, ifeat, W)
    cu, ci = _norms(du_parts, di_parts)
    s_all, giu, gii = _edge_scales(src_all, dst_all, et_all, cu, ci)
    iagg_p, uagg_p = _aggregate(mu, mi, giu, gii, dst_all, src_all, s_all)
    iagg, uagg = iagg_p[:NII], uagg_p[:NUU]
    uo, io = _fc(uagg, iagg, fc_w, fc_b.reshape(1, ODIM))
    pred = _decode(uo, io, du_all, di_all)
    return pred.reshape(EDP)[:EDD][:, None]
